# baseline (speedup 1.0000x reference)
import functools

import jax
import jax.numpy as jnp
from jax import lax
from jax.experimental import pallas as pl
from jax.experimental.pallas import tpu as pltpu
from jax.experimental.pallas import tpu_sc as plsc

N = 320000
D = 128
S = 512
NC = 2
NS = 16
NW = NC * NS
CHUNK = 80
CHUNKS_PER_W = 85
ROWS_PER_W = CHUNKS_PER_W * CHUNK
N_SC = NW * ROWS_PER_W
TB = 256
TC_BLOCKS = (N - N_SC) // TB
TC_OFF_BLOCKS = N_SC // TB
CHUNKS_PAD = 128
ZROWS = S // NS
NBUF = 5
NGROUPS = CHUNKS_PER_W // NBUF
NL = 16
NV = D // NL
RUNROLL = 4

_mesh = plsc.VectorSubcoreMesh(core_axis_name="c", subcore_axis_name="s")


@functools.partial(
    pl.kernel,
    mesh=_mesh,
    out_type=jax.ShapeDtypeStruct((NC, S, D), jnp.float32),
    scratch_types=[
        pltpu.VMEM((NBUF, CHUNK, D), jnp.float32),
        pltpu.VMEM((CHUNKS_PAD, CHUNK), jnp.int32),
        pltpu.VMEM((8, CHUNKS_PAD), jnp.int32),
        pltpu.VMEM((CHUNKS_PAD, D), jnp.float32),
        pltpu.VMEM_SHARED((S, D), jnp.float32),
        pltpu.SemaphoreType.DMA((NBUF,)),
        pltpu.SemaphoreType.DMA((NBUF,)),
    ],
)
def _segsum_sc(h_hbm, idx_hbm, fidx_hbm, out_hbm, hbuf, ibuf, fbuf, stage, acc,
               lsem, ssem):
    cid = lax.axis_index("c")
    sid = lax.axis_index("s")
    wid = cid * NS + sid
    base = wid * ROWS_PER_W

    zvec = jnp.zeros((NL,), jnp.float32)

    for r in range(CHUNKS_PER_W, CHUNKS_PAD):
        for c in range(NV):
            stage[r, pl.ds(c * NL, NL)] = zvec
    pltpu.sync_copy(
        stage.at[pl.ds(CHUNKS_PAD - ZROWS, ZROWS)],
        acc.at[pl.ds(sid * ZROWS, ZROWS)],
    )

    pltpu.sync_copy(idx_hbm.at[pl.ds(wid * CHUNKS_PAD, CHUNKS_PAD)], ibuf)
    pltpu.sync_copy(fidx_hbm.at[pl.ds(wid * 8, 8)], fbuf)

    plsc.subcore_barrier()

    def _load(k, b):
        return pltpu.make_async_copy(
            h_hbm.at[pl.ds(base + k * CHUNK, CHUNK)], hbuf.at[b], lsem.at[b]
        )

    def _scatter(k, b):
        return pltpu.make_async_copy(hbuf.at[b], acc.at[ibuf.at[k]], ssem.at[b])

    for b in range(NBUF):
        _load(b, b).start()

    def group(g, carry):
        k0 = g * NBUF
        for b in range(NBUF):
            k = k0 + b
            _load(k, b).wait()

            iv_first = ibuf[k, pl.ds(0, NL)]
            iv_last = ibuf[k, pl.ds(CHUNK - NL, NL)]
            single = iv_first[0] == iv_last[NL - 1]

            @pl.when(single)
            def _fast():
                def rows(r, accs):
                    out = list(accs)
                    for u in range(RUNROLL):
                        for c in range(NV):
                            out[c] = out[c] + hbuf[b, r * RUNROLL + u,
                                                   pl.ds(c * NL, NL)]
                    return tuple(out)

                sums = lax.fori_loop(
                    0, CHUNK // RUNROLL, rows,
                    tuple(jnp.zeros((NL,), jnp.float32) for _ in range(NV)),
                )
                for c in range(NV):
                    stage[k, pl.ds(c * NL, NL)] = sums[c]

            @pl.when(jnp.logical_not(single))
            def _slow():
                for c in range(NV):
                    stage[k, pl.ds(c * NL, NL)] = zvec
                _scatter(k, b).start(add=True)
                _scatter(k, b).wait()

            @pl.when(g != NGROUPS - 1)
            def _():
                _load(k + NBUF, b).start()

        return carry

    lax.fori_loop(0, NGROUPS, group, 0)

    pltpu.sync_copy(stage, acc.at[fbuf.at[0]], add=True)

    plsc.subcore_barrier()

    pltpu.sync_copy(
        acc.at[pl.ds(sid * ZROWS, ZROWS)],
        out_hbm.at[cid, pl.ds(sid * ZROWS, ZROWS)],
    )


def _tc_body(idx_ref, h_ref, o_ref):
    @pl.when(pl.program_id(0) == 0)
    def _():
        o_ref[...] = jnp.zeros_like(o_ref)

    idx = idx_ref[0, 0, :]
    onehot = (
        lax.broadcasted_iota(jnp.int32, (S, TB), 0) == idx[None, :]
    ).astype(jnp.float32)
    o_ref[...] += jnp.dot(
        onehot, h_ref[...],
        preferred_element_type=jnp.float32,
        precision=lax.Precision.HIGHEST,
    )


def _tc_segsum(h, idx_tc):
    return pl.pallas_call(
        _tc_body,
        grid=(TC_BLOCKS,),
        in_specs=[
            pl.BlockSpec((1, 1, TB), lambda i: (i, 0, 0)),
            pl.BlockSpec((TB, D), lambda i: (i + TC_OFF_BLOCKS, 0)),
        ],
        out_specs=pl.BlockSpec((S, D), lambda i: (0, 0)),
        out_shape=jax.ShapeDtypeStruct((S, D), jnp.float32),
    )(idx_tc, h)


def _merge_body(p_ref, t_ref, o_ref):
    o_ref[...] = p_ref[0] + p_ref[1] + t_ref[...]


def _merge(partials, tc_out):
    return pl.pallas_call(
        _merge_body,
        out_shape=jax.ShapeDtypeStruct((S, D), jnp.float32),
    )(partials, tc_out)


@jax.jit
def kernel(h, index):
    index = index.astype(jnp.int32)
    idx3d = index[:N_SC].reshape(NW, CHUNKS_PER_W, CHUNK)
    idx_pad = jnp.pad(idx3d, ((0, 0), (0, CHUNKS_PAD - CHUNKS_PER_W), (0, 0)))
    idx2d = idx_pad.reshape(NW * CHUNKS_PAD, CHUNK)
    fidx = jnp.pad(
        index[:N_SC:CHUNK].reshape(NW, 1, CHUNKS_PER_W),
        ((0, 0), (0, 7), (0, CHUNKS_PAD - CHUNKS_PER_W)),
    ).reshape(NW * 8, CHUNKS_PAD)
    idx_tc = index[N_SC:].reshape(TC_BLOCKS, 1, TB)
    partials = _segsum_sc(h, idx2d, fidx)
    tc_out = _tc_segsum(h, idx_tc)
    return _merge(partials, tc_out)


# speedup vs baseline: 1.2662x; 1.2662x over previous
import functools

import jax
import jax.numpy as jnp
from jax import lax
from jax.experimental import pallas as pl
from jax.experimental.pallas import tpu as pltpu
from jax.experimental.pallas import tpu_sc as plsc

N = 320000
D = 128
S = 512
NC = 2
NS = 16
NW = NC * NS
CHUNK = 80
CHUNKS_PER_W = 85
ROWS_PER_W = CHUNKS_PER_W * CHUNK
N_SC = NW * ROWS_PER_W
TB = 256
TC_BLOCKS = (N - N_SC) // TB
TC_OFF_BLOCKS = N_SC // TB
CHUNKS_PAD = 128
ZROWS = S // NS
NBUF = 5
NGROUPS = CHUNKS_PER_W // NBUF
NL = 16
NV = D // NL
RUNROLL = 4

_mesh = plsc.VectorSubcoreMesh(core_axis_name="c", subcore_axis_name="s")


@functools.partial(
    pl.kernel,
    mesh=_mesh,
    out_type=jax.ShapeDtypeStruct((NC, S, D), jnp.float32),
    scratch_types=[
        pltpu.VMEM((NBUF, CHUNK, D), jnp.float32),
        pltpu.VMEM((CHUNKS_PAD, CHUNK), jnp.int32),
        pltpu.VMEM((8, CHUNKS_PAD), jnp.int32),
        pltpu.VMEM((CHUNKS_PAD, D), jnp.float32),
        pltpu.VMEM_SHARED((S, D), jnp.float32),
        pltpu.SemaphoreType.DMA((NBUF,)),
        pltpu.SemaphoreType.DMA((NBUF,)),
    ],
)
def _segsum_sc(h_hbm, idx_hbm, fidx_hbm, out_hbm, hbuf, ibuf, fbuf, stage, acc,
               lsem, ssem):
    cid = lax.axis_index("c")
    sid = lax.axis_index("s")
    wid = cid * NS + sid
    base = wid * ROWS_PER_W

    zvec = jnp.zeros((NL,), jnp.float32)

    for r in range(CHUNKS_PER_W, CHUNKS_PAD):
        for c in range(NV):
            stage[r, pl.ds(c * NL, NL)] = zvec
    pltpu.sync_copy(
        stage.at[pl.ds(CHUNKS_PAD - ZROWS, ZROWS)],
        acc.at[pl.ds(sid * ZROWS, ZROWS)],
    )

    pltpu.sync_copy(idx_hbm.at[pl.ds(wid * CHUNKS_PAD, CHUNKS_PAD)], ibuf)
    pltpu.sync_copy(fidx_hbm.at[pl.ds(wid * 8, 8)], fbuf)

    plsc.subcore_barrier()

    def _load(k, b):
        return pltpu.make_async_copy(
            h_hbm.at[pl.ds(base + k * CHUNK, CHUNK)], hbuf.at[b], lsem.at[b]
        )

    def _scatter(k, b):
        return pltpu.make_async_copy(hbuf.at[b], acc.at[ibuf.at[k]], ssem.at[b])

    for b in range(NBUF):
        _load(b, b).start()

    def group(g, carry):
        k0 = g * NBUF
        for b in range(NBUF):
            k = k0 + b
            _load(k, b).wait()

            iv_first = ibuf[k, pl.ds(0, NL)]
            iv_last = ibuf[k, pl.ds(CHUNK - NL, NL)]
            single = iv_first[0] == iv_last[NL - 1]

            @pl.when(single)
            def _fast():
                def rows(r, accs):
                    out = list(accs)
                    for u in range(RUNROLL):
                        for c in range(NV):
                            out[c] = out[c] + hbuf[b, r * RUNROLL + u,
                                                   pl.ds(c * NL, NL)]
                    return tuple(out)

                sums = lax.fori_loop(
                    0, CHUNK // RUNROLL, rows,
                    tuple(jnp.zeros((NL,), jnp.float32) for _ in range(NV)),
                )
                for c in range(NV):
                    stage[k, pl.ds(c * NL, NL)] = sums[c]

            @pl.when(jnp.logical_not(single))
            def _slow():
                for c in range(NV):
                    stage[k, pl.ds(c * NL, NL)] = zvec
                _scatter(k, b).start(add=True)
                _scatter(k, b).wait()

            @pl.when(g != NGROUPS - 1)
            def _():
                _load(k + NBUF, b).start()

        return carry

    lax.fori_loop(0, NGROUPS, group, 0)

    pltpu.sync_copy(stage, acc.at[fbuf.at[0]], add=True)

    plsc.subcore_barrier()

    pltpu.sync_copy(
        acc.at[pl.ds(sid * ZROWS, ZROWS)],
        out_hbm.at[cid, pl.ds(sid * ZROWS, ZROWS)],
    )


def _tc_body(idx_ref, h_ref, o_ref):
    @pl.when(pl.program_id(0) == 0)
    def _():
        o_ref[...] = jnp.zeros_like(o_ref)

    idx = idx_ref[0, 0, :]
    onehot = (
        lax.broadcasted_iota(jnp.int32, (S, TB), 0) == idx[None, :]
    ).astype(jnp.float32)
    o_ref[...] += jnp.dot(
        onehot, h_ref[...],
        preferred_element_type=jnp.float32,
        precision=lax.Precision.DEFAULT,
    )


def _tc_segsum(h, idx_tc):
    return pl.pallas_call(
        _tc_body,
        grid=(TC_BLOCKS,),
        in_specs=[
            pl.BlockSpec((1, 1, TB), lambda i: (i, 0, 0)),
            pl.BlockSpec((TB, D), lambda i: (i + TC_OFF_BLOCKS, 0)),
        ],
        out_specs=pl.BlockSpec((S, D), lambda i: (0, 0)),
        out_shape=jax.ShapeDtypeStruct((S, D), jnp.float32),
    )(idx_tc, h)


def _merge_body(p_ref, t_ref, o_ref):
    o_ref[...] = p_ref[0] + p_ref[1] + t_ref[...]


def _merge(partials, tc_out):
    return pl.pallas_call(
        _merge_body,
        out_shape=jax.ShapeDtypeStruct((S, D), jnp.float32),
    )(partials, tc_out)


@jax.jit
def kernel(h, index):
    index = index.astype(jnp.int32)
    idx3d = index[:N_SC].reshape(NW, CHUNKS_PER_W, CHUNK)
    idx_pad = jnp.pad(idx3d, ((0, 0), (0, CHUNKS_PAD - CHUNKS_PER_W), (0, 0)))
    idx2d = idx_pad.reshape(NW * CHUNKS_PAD, CHUNK)
    fidx = jnp.pad(
        index[:N_SC:CHUNK].reshape(NW, 1, CHUNKS_PER_W),
        ((0, 0), (0, 7), (0, CHUNKS_PAD - CHUNKS_PER_W)),
    ).reshape(NW * 8, CHUNKS_PAD)
    idx_tc = index[N_SC:].reshape(TC_BLOCKS, 1, TB)
    partials = _segsum_sc(h, idx2d, fidx)
    tc_out = _tc_segsum(h, idx_tc)
    return _merge(partials, tc_out)


# speedup vs baseline: 3.9943x; 3.1547x over previous
import functools

import jax
import jax.numpy as jnp
from jax import lax
from jax.experimental import pallas as pl
from jax.experimental.pallas import tpu as pltpu
from jax.experimental.pallas import tpu_sc as plsc

N = 320000
D = 128
S = 512
NC = 2
NS = 16
NW = NC * NS
ROWS_PER_W = N // NW
CHUNK = 80
CHUNKS_PER_W = ROWS_PER_W // CHUNK
CHUNKS_PAD = 128
ZROWS = S // NS
NBUF = 8
NL = 16
NV = D // NL
RUNROLL = 4

_mesh = plsc.VectorSubcoreMesh(core_axis_name="c", subcore_axis_name="s")


@functools.partial(
    pl.kernel,
    mesh=_mesh,
    out_type=jax.ShapeDtypeStruct((NC, S, D), jnp.float32),
    scratch_types=[
        pltpu.VMEM((NBUF, CHUNK, D), jnp.float32),
        pltpu.VMEM((CHUNKS_PAD, CHUNK), jnp.int32),
        pltpu.VMEM((8, CHUNKS_PAD), jnp.int32),
        pltpu.VMEM((CHUNKS_PAD, D), jnp.float32),
        pltpu.VMEM_SHARED((S, D), jnp.float32),
        pltpu.SemaphoreType.DMA((NBUF,)),
        pltpu.SemaphoreType.DMA((NBUF,)),
    ],
)
def _segsum_sc(h_hbm, idx_hbm, fidx_hbm, out_hbm, hbuf, ibuf, fbuf, stage, acc,
               lsem, ssem):
    cid = lax.axis_index("c")
    sid = lax.axis_index("s")
    wid = cid * NS + sid
    base = wid * ROWS_PER_W

    def _load(k, b):
        return pltpu.make_async_copy(
            h_hbm.at[pl.ds(base + k * CHUNK, CHUNK)], hbuf.at[b], lsem.at[b]
        )

    def _scatter(k, b):
        return pltpu.make_async_copy(hbuf.at[b], acc.at[ibuf.at[k]], ssem.at[b])

    for b in range(NBUF):
        _load(b, b).start()

    zvec = jnp.zeros((NL,), jnp.float32)

    for r in range(ZROWS):
        for c in range(NV):
            stage[CHUNKS_PER_W - ZROWS + 3 + r, pl.ds(c * NL, NL)] = zvec
    pltpu.sync_copy(
        stage.at[pl.ds(CHUNKS_PER_W - ZROWS + 3, ZROWS)],
        acc.at[pl.ds(sid * ZROWS, ZROWS)],
    )

    pltpu.sync_copy(idx_hbm.at[pl.ds(wid * CHUNKS_PAD, CHUNKS_PAD)], ibuf)
    pltpu.sync_copy(fidx_hbm.at[pl.ds(wid * 8, 8)], fbuf)

    plsc.subcore_barrier()

    def chunk_step(k, carry):
        b = lax.bitwise_and(k, NBUF - 1)
        _load(k, b).wait()

        iv_first = ibuf[k, pl.ds(0, NL)]
        iv_last = ibuf[k, pl.ds(CHUNK - NL, NL)]
        single = iv_first[0] == iv_last[NL - 1]

        @pl.when(single)
        def _fast():
            def rows(r, accs):
                out = list(accs)
                for u in range(RUNROLL):
                    for c in range(NV):
                        out[c] = out[c] + hbuf[b, r * RUNROLL + u,
                                               pl.ds(c * NL, NL)]
                return tuple(out)

            sums = lax.fori_loop(
                0, CHUNK // RUNROLL, rows,
                tuple(jnp.zeros((NL,), jnp.float32) for _ in range(NV)),
            )
            for c in range(NV):
                stage[k, pl.ds(c * NL, NL)] = sums[c]

        @pl.when(jnp.logical_not(single))
        def _slow():
            for c in range(NV):
                stage[k, pl.ds(c * NL, NL)] = zvec
            _scatter(k, b).start(add=True)
            _scatter(k, b).wait()

        @pl.when(k < CHUNKS_PER_W - NBUF)
        def _():
            _load(k + NBUF, b).start()

        return carry

    lax.fori_loop(0, CHUNKS_PER_W, chunk_step, 0)

    pltpu.sync_copy(stage, acc.at[fbuf.at[0]], add=True)

    plsc.subcore_barrier()

    pltpu.sync_copy(
        acc.at[pl.ds(sid * ZROWS, ZROWS)],
        out_hbm.at[cid, pl.ds(sid * ZROWS, ZROWS)],
    )


def _merge_body(p_ref, o_ref):
    o_ref[...] = p_ref[0] + p_ref[1]


def _merge(partials):
    return pl.pallas_call(
        _merge_body,
        out_shape=jax.ShapeDtypeStruct((S, D), jnp.float32),
    )(partials)


@jax.jit
def kernel(h, index):
    index = index.astype(jnp.int32)
    idx3d = index.reshape(NW, CHUNKS_PER_W, CHUNK)
    idx_pad = jnp.pad(idx3d, ((0, 0), (0, CHUNKS_PAD - CHUNKS_PER_W), (0, 0)))
    idx2d = idx_pad.reshape(NW * CHUNKS_PAD, CHUNK)
    fidx = jnp.pad(
        index[::CHUNK].reshape(NW, 1, CHUNKS_PER_W),
        ((0, 0), (0, 7), (0, CHUNKS_PAD - CHUNKS_PER_W)),
    ).reshape(NW * 8, CHUNKS_PAD)
    partials = _segsum_sc(h, idx2d, fidx)
    return _merge(partials)
